# Initial kernel scaffold; baseline (speedup 1.0000x reference)
import jax
import jax.numpy as jnp
from jax.experimental import pallas as pl


def kernel(x, adj, batch, W1, b1, W2, b2, Wc1, bc1, Wc2, bc2):
    raise NotImplementedError("write your pallas kernel here")


# speedup vs baseline: 1.5322x; 1.5322x over previous
import jax
import jax.numpy as jnp
from jax.experimental import pallas as pl
from jax.experimental.pallas import tpu as pltpu

_BM = 512
_NUM_GRAPHS = 64


def _prep_body(adj_ref, x_ref, w1_ref, abf_ref, dinv_ref, hs1_ref):
    a = adj_ref[...]
    abf_ref[...] = a.astype(jnp.bfloat16)
    deg = jnp.sum(a, axis=1, keepdims=True) + 1.0
    dinv = jax.lax.rsqrt(deg)
    dinv_ref[...] = jnp.broadcast_to(dinv, dinv_ref.shape)
    xw = jnp.dot(x_ref[...], w1_ref[...], preferred_element_type=jnp.float32)
    hs1_ref[...] = (xw * dinv).astype(jnp.bfloat16)


def _gcn_mid_body(abf_ref, hsf_ref, hsb_ref, dinv_ref, w2_ref, b1_ref, out_ref):
    dinv = dinv_ref[...][:, :1]
    t = jnp.dot(abf_ref[...], hsf_ref[...], preferred_element_type=jnp.float32)
    h = jnp.maximum(dinv * (t + hsb_ref[...].astype(jnp.float32)) + b1_ref[...], 0.0)
    out_ref[...] = (dinv * jnp.dot(h.astype(jnp.bfloat16), w2_ref[...],
                                   preferred_element_type=jnp.float32)
                    ).astype(jnp.bfloat16)


def _gcn_out_body(abf_ref, hsf_ref, hsb_ref, dinv_ref, b2_ref, out_ref):
    dinv = dinv_ref[...][:, :1]
    t = jnp.dot(abf_ref[...], hsf_ref[...], preferred_element_type=jnp.float32)
    out_ref[...] = dinv * (t + hsb_ref[...].astype(jnp.float32)) + b2_ref[...]


def _pool_head_body(h2_ref, batch_ref, wc1_ref, bc1_ref, wc2_ref, bc2_ref, out_ref):
    g = out_ref.shape[0]
    n = batch_ref.shape[1]
    seg = jax.lax.broadcasted_iota(jnp.int32, (g, n), 0)
    onehot = jnp.where(batch_ref[...] == seg, 1.0, 0.0)
    cnt = jnp.sum(onehot, axis=1, keepdims=True)
    sums = jnp.dot(onehot, h2_ref[...], preferred_element_type=jnp.float32)
    pooled = sums / jnp.maximum(cnt, 1.0)
    z = jnp.maximum(jnp.dot(pooled, wc1_ref[...], preferred_element_type=jnp.float32)
                    + bc1_ref[...], 0.0)
    out_ref[...] = (jnp.dot(z, wc2_ref[...], preferred_element_type=jnp.float32)
                    + bc2_ref[...])


def kernel(x, adj, batch, W1, b1, W2, b2, Wc1, bc1, Wc2, bc2):
    n, din = x.shape
    dh = W1.shape[1]
    dout = W2.shape[1]
    ncls = Wc2.shape[1]
    g = _NUM_GRAPHS
    bm = _BM
    r = n // bm

    batch2 = batch.astype(jnp.int32).reshape(1, n)
    b1_2 = b1.reshape(1, dh)
    b2_2 = b2.reshape(1, dout)
    bc1_2 = bc1.reshape(1, dh)
    bc2_2 = bc2.reshape(1, ncls)
    w2b = W2.astype(jnp.bfloat16)

    par = pltpu.CompilerParams(dimension_semantics=("parallel",),
                               vmem_limit_bytes=56 * 1024 * 1024)

    abf, dinvb, hs1 = pl.pallas_call(
        _prep_body,
        grid=(r,),
        in_specs=[
            pl.BlockSpec((bm, n), lambda i: (i, 0)),
            pl.BlockSpec((bm, din), lambda i: (i, 0)),
            pl.BlockSpec((din, dh), lambda i: (0, 0)),
        ],
        out_specs=[
            pl.BlockSpec((bm, n), lambda i: (i, 0)),
            pl.BlockSpec((bm, 128), lambda i: (i, 0)),
            pl.BlockSpec((bm, dh), lambda i: (i, 0)),
        ],
        out_shape=[
            jax.ShapeDtypeStruct((n, n), jnp.bfloat16),
            jax.ShapeDtypeStruct((n, 128), jnp.float32),
            jax.ShapeDtypeStruct((n, dh), jnp.bfloat16),
        ],
        compiler_params=par,
        name="gcn_prep",
    )(adj, x, W1)

    hs2 = pl.pallas_call(
        _gcn_mid_body,
        grid=(r,),
        in_specs=[
            pl.BlockSpec((bm, n), lambda i: (i, 0)),
            pl.BlockSpec((n, dh), lambda i: (0, 0)),
            pl.BlockSpec((bm, dh), lambda i: (i, 0)),
            pl.BlockSpec((bm, 128), lambda i: (i, 0)),
            pl.BlockSpec((dh, dout), lambda i: (0, 0)),
            pl.BlockSpec((1, dh), lambda i: (0, 0)),
        ],
        out_specs=pl.BlockSpec((bm, dout), lambda i: (i, 0)),
        out_shape=jax.ShapeDtypeStruct((n, dout), jnp.bfloat16),
        compiler_params=par,
        name="gcn_layer1",
    )(abf, hs1, hs1, dinvb, w2b, b1_2)

    h2 = pl.pallas_call(
        _gcn_out_body,
        grid=(r,),
        in_specs=[
            pl.BlockSpec((bm, n), lambda i: (i, 0)),
            pl.BlockSpec((n, dout), lambda i: (0, 0)),
            pl.BlockSpec((bm, dout), lambda i: (i, 0)),
            pl.BlockSpec((bm, 128), lambda i: (i, 0)),
            pl.BlockSpec((1, dout), lambda i: (0, 0)),
        ],
        out_specs=pl.BlockSpec((bm, dout), lambda i: (i, 0)),
        out_shape=jax.ShapeDtypeStruct((n, dout), jnp.float32),
        compiler_params=par,
        name="gcn_layer2",
    )(abf, hs2, hs2, dinvb, b2_2)

    out = pl.pallas_call(
        _pool_head_body,
        grid=(1,),
        in_specs=[
            pl.BlockSpec((n, dout), lambda i: (0, 0)),
            pl.BlockSpec((1, n), lambda i: (0, 0)),
            pl.BlockSpec((dout, dh), lambda i: (0, 0)),
            pl.BlockSpec((1, dh), lambda i: (0, 0)),
            pl.BlockSpec((dh, ncls), lambda i: (0, 0)),
            pl.BlockSpec((1, ncls), lambda i: (0, 0)),
        ],
        out_specs=pl.BlockSpec((g, ncls), lambda i: (0, 0)),
        out_shape=jax.ShapeDtypeStruct((g, ncls), jnp.float32),
        compiler_params=pltpu.CompilerParams(
            dimension_semantics=("arbitrary",),
            vmem_limit_bytes=56 * 1024 * 1024),
        name="pool_head",
    )(h2, batch2, Wc1, bc1_2, Wc2, bc2_2)
    return out


# speedup vs baseline: 1.7612x; 1.1495x over previous
import jax
import jax.numpy as jnp
from jax.experimental import pallas as pl
from jax.experimental.pallas import tpu as pltpu

_BM = 512
_NUM_GRAPHS = 64


def _prep_body(adj_ref, x_ref, w1_ref, ai8_ref, dinv_ref, hs1_ref):
    a = adj_ref[...]
    ai8_ref[...] = a.astype(jnp.int8)
    deg = jnp.sum(a, axis=1, keepdims=True) + 1.0
    dinv = jax.lax.rsqrt(deg)
    dinv_ref[...] = jnp.broadcast_to(dinv, dinv_ref.shape)
    xw = jnp.dot(x_ref[...], w1_ref[...], preferred_element_type=jnp.float32)
    hs1_ref[...] = (xw * dinv).astype(jnp.bfloat16)


def _gcn_mid_body(ai8_ref, hsf_ref, hsb_ref, dinv_ref, w2_ref, b1_ref, out_ref):
    dinv = dinv_ref[...][:, :1]
    a = ai8_ref[...].astype(jnp.bfloat16)
    t = jnp.dot(a, hsf_ref[...], preferred_element_type=jnp.float32)
    h = jnp.maximum(dinv * (t + hsb_ref[...].astype(jnp.float32)) + b1_ref[...], 0.0)
    out_ref[...] = (dinv * jnp.dot(h.astype(jnp.bfloat16), w2_ref[...],
                                   preferred_element_type=jnp.float32)
                    ).astype(jnp.bfloat16)


def _gcn_out_body(ai8_ref, hsf_ref, hsb_ref, dinv_ref, b2_ref, out_ref):
    dinv = dinv_ref[...][:, :1]
    a = ai8_ref[...].astype(jnp.bfloat16)
    t = jnp.dot(a, hsf_ref[...], preferred_element_type=jnp.float32)
    out_ref[...] = (dinv * (t + hsb_ref[...].astype(jnp.float32))
                    + b2_ref[...]).astype(jnp.bfloat16)


def _pool_head_body(h2_ref, batch_ref, wc1_ref, bc1_ref, wc2_ref, bc2_ref, out_ref):
    g = out_ref.shape[0]
    n = batch_ref.shape[1]
    seg = jax.lax.broadcasted_iota(jnp.int32, (g, n), 0)
    onehot = jnp.where(batch_ref[...] == seg, 1.0, 0.0)
    cnt = jnp.sum(onehot, axis=1, keepdims=True)
    sums = jnp.dot(onehot.astype(jnp.bfloat16), h2_ref[...],
                   preferred_element_type=jnp.float32)
    pooled = sums / jnp.maximum(cnt, 1.0)
    z = jnp.maximum(jnp.dot(pooled, wc1_ref[...], preferred_element_type=jnp.float32)
                    + bc1_ref[...], 0.0)
    out_ref[...] = (jnp.dot(z, wc2_ref[...], preferred_element_type=jnp.float32)
                    + bc2_ref[...])


def kernel(x, adj, batch, W1, b1, W2, b2, Wc1, bc1, Wc2, bc2):
    n, din = x.shape
    dh = W1.shape[1]
    dout = W2.shape[1]
    ncls = Wc2.shape[1]
    g = _NUM_GRAPHS
    bm = _BM
    r = n // bm

    batch2 = batch.astype(jnp.int32).reshape(1, n)
    b1_2 = b1.reshape(1, dh)
    b2_2 = b2.reshape(1, dout)
    bc1_2 = bc1.reshape(1, dh)
    bc2_2 = bc2.reshape(1, ncls)
    w2b = W2.astype(jnp.bfloat16)

    par = pltpu.CompilerParams(dimension_semantics=("parallel",),
                               vmem_limit_bytes=56 * 1024 * 1024)

    abf, dinvb, hs1 = pl.pallas_call(
        _prep_body,
        grid=(r,),
        in_specs=[
            pl.BlockSpec((bm, n), lambda i: (i, 0)),
            pl.BlockSpec((bm, din), lambda i: (i, 0)),
            pl.BlockSpec((din, dh), lambda i: (0, 0)),
        ],
        out_specs=[
            pl.BlockSpec((bm, n), lambda i: (i, 0)),
            pl.BlockSpec((bm, 128), lambda i: (i, 0)),
            pl.BlockSpec((bm, dh), lambda i: (i, 0)),
        ],
        out_shape=[
            jax.ShapeDtypeStruct((n, n), jnp.int8),
            jax.ShapeDtypeStruct((n, 128), jnp.float32),
            jax.ShapeDtypeStruct((n, dh), jnp.bfloat16),
        ],
        compiler_params=par,
        name="gcn_prep",
    )(adj, x, W1)

    hs2 = pl.pallas_call(
        _gcn_mid_body,
        grid=(r,),
        in_specs=[
            pl.BlockSpec((bm, n), lambda i: (i, 0)),
            pl.BlockSpec((n, dh), lambda i: (0, 0)),
            pl.BlockSpec((bm, dh), lambda i: (i, 0)),
            pl.BlockSpec((bm, 128), lambda i: (i, 0)),
            pl.BlockSpec((dh, dout), lambda i: (0, 0)),
            pl.BlockSpec((1, dh), lambda i: (0, 0)),
        ],
        out_specs=pl.BlockSpec((bm, dout), lambda i: (i, 0)),
        out_shape=jax.ShapeDtypeStruct((n, dout), jnp.bfloat16),
        compiler_params=par,
        name="gcn_layer1",
    )(abf, hs1, hs1, dinvb, w2b, b1_2)

    h2 = pl.pallas_call(
        _gcn_out_body,
        grid=(r,),
        in_specs=[
            pl.BlockSpec((bm, n), lambda i: (i, 0)),
            pl.BlockSpec((n, dout), lambda i: (0, 0)),
            pl.BlockSpec((bm, dout), lambda i: (i, 0)),
            pl.BlockSpec((bm, 128), lambda i: (i, 0)),
            pl.BlockSpec((1, dout), lambda i: (0, 0)),
        ],
        out_specs=pl.BlockSpec((bm, dout), lambda i: (i, 0)),
        out_shape=jax.ShapeDtypeStruct((n, dout), jnp.bfloat16),
        compiler_params=par,
        name="gcn_layer2",
    )(abf, hs2, hs2, dinvb, b2_2)

    out = pl.pallas_call(
        _pool_head_body,
        grid=(1,),
        in_specs=[
            pl.BlockSpec((n, dout), lambda i: (0, 0)),
            pl.BlockSpec((1, n), lambda i: (0, 0)),
            pl.BlockSpec((dout, dh), lambda i: (0, 0)),
            pl.BlockSpec((1, dh), lambda i: (0, 0)),
            pl.BlockSpec((dh, ncls), lambda i: (0, 0)),
            pl.BlockSpec((1, ncls), lambda i: (0, 0)),
        ],
        out_specs=pl.BlockSpec((g, ncls), lambda i: (0, 0)),
        out_shape=jax.ShapeDtypeStruct((g, ncls), jnp.float32),
        compiler_params=pltpu.CompilerParams(
            dimension_semantics=("arbitrary",),
            vmem_limit_bytes=56 * 1024 * 1024),
        name="pool_head",
    )(h2, batch2, Wc1, bc1_2, Wc2, bc2_2)
    return out


# speedup vs baseline: 1.7696x; 1.0048x over previous
import jax
import jax.numpy as jnp
from jax.experimental import pallas as pl
from jax.experimental.pallas import tpu as pltpu

_BM1 = 512
_BM = 1024
_NUM_GRAPHS = 64


def _prep_body(adj_ref, x_ref, w1_ref, ai8_ref, dinv_ref, hs1_ref):
    a = adj_ref[...]
    ai8_ref[...] = a.astype(jnp.int8)
    deg = jnp.sum(a, axis=1, keepdims=True) + 1.0
    dinv = jax.lax.rsqrt(deg)
    dinv_ref[...] = jnp.broadcast_to(dinv, dinv_ref.shape)
    xw = jnp.dot(x_ref[...], w1_ref[...], preferred_element_type=jnp.float32)
    hs1_ref[...] = (xw * dinv).astype(jnp.bfloat16)


def _gcn_mid_body(ai8_ref, hsf_ref, hsb_ref, dinv_ref, w2_ref, b1_ref, out_ref):
    dinv = dinv_ref[...][:, :1]
    a = ai8_ref[...].astype(jnp.bfloat16)
    t = jnp.dot(a, hsf_ref[...], preferred_element_type=jnp.float32)
    h = jnp.maximum(dinv * (t + hsb_ref[...].astype(jnp.float32)) + b1_ref[...], 0.0)
    out_ref[...] = (dinv * jnp.dot(h.astype(jnp.bfloat16), w2_ref[...],
                                   preferred_element_type=jnp.float32)
                    ).astype(jnp.bfloat16)


def _gcn_out_body(ai8_ref, hsf_ref, hsb_ref, dinv_ref, b2_ref, batch_ref,
                  wc1_ref, bc1_ref, wc2_ref, bc2_ref, out_ref,
                  psum_ref, pcnt_ref):
    i = pl.program_id(0)
    nsteps = pl.num_programs(0)
    g = psum_ref.shape[0]
    bm = ai8_ref.shape[0]

    @pl.when(i == 0)
    def _():
        psum_ref[...] = jnp.zeros_like(psum_ref)
        pcnt_ref[...] = jnp.zeros_like(pcnt_ref)

    dinv = dinv_ref[...][:, :1]
    a = ai8_ref[...].astype(jnp.bfloat16)
    t = jnp.dot(a, hsf_ref[...], preferred_element_type=jnp.float32)
    h2 = dinv * (t + hsb_ref[...].astype(jnp.float32)) + b2_ref[...]

    seg = jax.lax.broadcasted_iota(jnp.int32, (g, bm), 0)
    onehot = jnp.where(batch_ref[...] == seg, 1.0, 0.0)
    psum_ref[...] += jnp.dot(onehot.astype(jnp.bfloat16), h2.astype(jnp.bfloat16),
                             preferred_element_type=jnp.float32)
    pcnt_ref[...] += jnp.broadcast_to(
        jnp.sum(onehot, axis=1, keepdims=True), pcnt_ref.shape)

    @pl.when(i == nsteps - 1)
    def _():
        pooled = psum_ref[...] / jnp.maximum(pcnt_ref[...][:, :1], 1.0)
        z = jnp.maximum(
            jnp.dot(pooled, wc1_ref[...], preferred_element_type=jnp.float32)
            + bc1_ref[...], 0.0)
        out_ref[...] = (jnp.dot(z, wc2_ref[...], preferred_element_type=jnp.float32)
                        + bc2_ref[...])


def kernel(x, adj, batch, W1, b1, W2, b2, Wc1, bc1, Wc2, bc2):
    n, din = x.shape
    dh = W1.shape[1]
    dout = W2.shape[1]
    ncls = Wc2.shape[1]
    g = _NUM_GRAPHS
    r1 = n // _BM1
    r = n // _BM

    batch2 = batch.astype(jnp.int32).reshape(1, n)
    b1_2 = b1.reshape(1, dh)
    b2_2 = b2.reshape(1, dout)
    bc1_2 = bc1.reshape(1, dh)
    bc2_2 = bc2.reshape(1, ncls)
    w2b = W2.astype(jnp.bfloat16)

    par = pltpu.CompilerParams(dimension_semantics=("parallel",),
                               vmem_limit_bytes=56 * 1024 * 1024)

    ai8, dinvb, hs1 = pl.pallas_call(
        _prep_body,
        grid=(r1,),
        in_specs=[
            pl.BlockSpec((_BM1, n), lambda i: (i, 0)),
            pl.BlockSpec((_BM1, din), lambda i: (i, 0)),
            pl.BlockSpec((din, dh), lambda i: (0, 0)),
        ],
        out_specs=[
            pl.BlockSpec((_BM1, n), lambda i: (i, 0)),
            pl.BlockSpec((_BM1, 128), lambda i: (i, 0)),
            pl.BlockSpec((_BM1, dh), lambda i: (i, 0)),
        ],
        out_shape=[
            jax.ShapeDtypeStruct((n, n), jnp.int8),
            jax.ShapeDtypeStruct((n, 128), jnp.float32),
            jax.ShapeDtypeStruct((n, dh), jnp.bfloat16),
        ],
        compiler_params=par,
        name="gcn_prep",
    )(adj, x, W1)

    hs2 = pl.pallas_call(
        _gcn_mid_body,
        grid=(r,),
        in_specs=[
            pl.BlockSpec((_BM, n), lambda i: (i, 0)),
            pl.BlockSpec((n, dh), lambda i: (0, 0)),
            pl.BlockSpec((_BM, dh), lambda i: (i, 0)),
            pl.BlockSpec((_BM, 128), lambda i: (i, 0)),
            pl.BlockSpec((dh, dout), lambda i: (0, 0)),
            pl.BlockSpec((1, dh), lambda i: (0, 0)),
        ],
        out_specs=pl.BlockSpec((_BM, dout), lambda i: (i, 0)),
        out_shape=jax.ShapeDtypeStruct((n, dout), jnp.bfloat16),
        compiler_params=par,
        name="gcn_layer1",
    )(ai8, hs1, hs1, dinvb, w2b, b1_2)

    out = pl.pallas_call(
        _gcn_out_body,
        grid=(r,),
        in_specs=[
            pl.BlockSpec((_BM, n), lambda i: (i, 0)),
            pl.BlockSpec((n, dout), lambda i: (0, 0)),
            pl.BlockSpec((_BM, dout), lambda i: (i, 0)),
            pl.BlockSpec((_BM, 128), lambda i: (i, 0)),
            pl.BlockSpec((1, dout), lambda i: (0, 0)),
            pl.BlockSpec((1, _BM), lambda i: (0, i)),
            pl.BlockSpec((dout, dh), lambda i: (0, 0)),
            pl.BlockSpec((1, dh), lambda i: (0, 0)),
            pl.BlockSpec((dh, ncls), lambda i: (0, 0)),
            pl.BlockSpec((1, ncls), lambda i: (0, 0)),
        ],
        out_specs=pl.BlockSpec((g, ncls), lambda i: (0, 0)),
        out_shape=jax.ShapeDtypeStruct((g, ncls), jnp.float32),
        scratch_shapes=[
            pltpu.VMEM((g, dout), jnp.float32),
            pltpu.VMEM((g, 128), jnp.float32),
        ],
        compiler_params=pltpu.CompilerParams(
            dimension_semantics=("arbitrary",),
            vmem_limit_bytes=56 * 1024 * 1024),
        name="gcn_layer2_pool",
    )(ai8, hs2, hs2, dinvb, b2_2, batch2, Wc1, bc1_2, Wc2, bc2_2)
    return out


# speedup vs baseline: 1.8892x; 1.0676x over previous
import jax
import jax.numpy as jnp
from jax.experimental import pallas as pl
from jax.experimental.pallas import tpu as pltpu

_BM1 = 512
_BM = 1024
_NUM_GRAPHS = 64


def _prep_body(adj_ref, x_ref, w1_ref, ai8_ref, dinv_ref, hs1_ref):
    a = adj_ref[...]
    ai8_ref[...] = a.astype(jnp.int8).astype(jnp.int4)
    deg = jnp.sum(a, axis=1, keepdims=True) + 1.0
    dinv = jax.lax.rsqrt(deg)
    dinv_ref[...] = jnp.broadcast_to(dinv, dinv_ref.shape)
    xw = jnp.dot(x_ref[...], w1_ref[...], preferred_element_type=jnp.float32)
    hs1_ref[...] = (xw * dinv).astype(jnp.bfloat16)


def _gcn_mid_body(ai8_ref, hsf_ref, hsb_ref, dinv_ref, w2_ref, b1_ref, out_ref):
    dinv = dinv_ref[...][:, :1]
    a = ai8_ref[...].astype(jnp.int8).astype(jnp.bfloat16)
    t = jnp.dot(a, hsf_ref[...], preferred_element_type=jnp.float32)
    h = jnp.maximum(dinv * (t + hsb_ref[...].astype(jnp.float32)) + b1_ref[...], 0.0)
    out_ref[...] = (dinv * jnp.dot(h.astype(jnp.bfloat16), w2_ref[...],
                                   preferred_element_type=jnp.float32)
                    ).astype(jnp.bfloat16)


def _gcn_out_body(ai8_ref, hsf_ref, hsb_ref, dinv_ref, b2_ref, batch_ref,
                  wc1_ref, bc1_ref, wc2_ref, bc2_ref, out_ref,
                  psum_ref, pcnt_ref):
    i = pl.program_id(0)
    nsteps = pl.num_programs(0)
    g = psum_ref.shape[0]
    bm = ai8_ref.shape[0]

    @pl.when(i == 0)
    def _():
        psum_ref[...] = jnp.zeros_like(psum_ref)
        pcnt_ref[...] = jnp.zeros_like(pcnt_ref)

    dinv = dinv_ref[...][:, :1]
    a = ai8_ref[...].astype(jnp.int8).astype(jnp.bfloat16)
    t = jnp.dot(a, hsf_ref[...], preferred_element_type=jnp.float32)
    h2 = dinv * (t + hsb_ref[...].astype(jnp.float32)) + b2_ref[...]

    seg = jax.lax.broadcasted_iota(jnp.int32, (g, bm), 0)
    onehot = jnp.where(batch_ref[...] == seg, 1.0, 0.0)
    psum_ref[...] += jnp.dot(onehot.astype(jnp.bfloat16), h2.astype(jnp.bfloat16),
                             preferred_element_type=jnp.float32)
    pcnt_ref[...] += jnp.broadcast_to(
        jnp.sum(onehot, axis=1, keepdims=True), pcnt_ref.shape)

    @pl.when(i == nsteps - 1)
    def _():
        pooled = psum_ref[...] / jnp.maximum(pcnt_ref[...][:, :1], 1.0)
        z = jnp.maximum(
            jnp.dot(pooled, wc1_ref[...], preferred_element_type=jnp.float32)
            + bc1_ref[...], 0.0)
        out_ref[...] = (jnp.dot(z, wc2_ref[...], preferred_element_type=jnp.float32)
                        + bc2_ref[...])


def kernel(x, adj, batch, W1, b1, W2, b2, Wc1, bc1, Wc2, bc2):
    n, din = x.shape
    dh = W1.shape[1]
    dout = W2.shape[1]
    ncls = Wc2.shape[1]
    g = _NUM_GRAPHS
    r1 = n // _BM1
    r = n // _BM

    batch2 = batch.astype(jnp.int32).reshape(1, n)
    b1_2 = b1.reshape(1, dh)
    b2_2 = b2.reshape(1, dout)
    bc1_2 = bc1.reshape(1, dh)
    bc2_2 = bc2.reshape(1, ncls)
    w2b = W2.astype(jnp.bfloat16)

    par = pltpu.CompilerParams(dimension_semantics=("parallel",),
                               vmem_limit_bytes=56 * 1024 * 1024)

    ai8, dinvb, hs1 = pl.pallas_call(
        _prep_body,
        grid=(r1,),
        in_specs=[
            pl.BlockSpec((_BM1, n), lambda i: (i, 0)),
            pl.BlockSpec((_BM1, din), lambda i: (i, 0)),
            pl.BlockSpec((din, dh), lambda i: (0, 0)),
        ],
        out_specs=[
            pl.BlockSpec((_BM1, n), lambda i: (i, 0)),
            pl.BlockSpec((_BM1, 8), lambda i: (i, 0)),
            pl.BlockSpec((_BM1, dh), lambda i: (i, 0)),
        ],
        out_shape=[
            jax.ShapeDtypeStruct((n, n), jnp.int4),
            jax.ShapeDtypeStruct((n, 8), jnp.float32),
            jax.ShapeDtypeStruct((n, dh), jnp.bfloat16),
        ],
        compiler_params=par,
        name="gcn_prep",
    )(adj, x, W1)

    hs2 = pl.pallas_call(
        _gcn_mid_body,
        grid=(r,),
        in_specs=[
            pl.BlockSpec((_BM, n), lambda i: (i, 0)),
            pl.BlockSpec((n, dh), lambda i: (0, 0)),
            pl.BlockSpec((_BM, dh), lambda i: (i, 0)),
            pl.BlockSpec((_BM, 8), lambda i: (i, 0)),
            pl.BlockSpec((dh, dout), lambda i: (0, 0)),
            pl.BlockSpec((1, dh), lambda i: (0, 0)),
        ],
        out_specs=pl.BlockSpec((_BM, dout), lambda i: (i, 0)),
        out_shape=jax.ShapeDtypeStruct((n, dout), jnp.bfloat16),
        compiler_params=par,
        name="gcn_layer1",
    )(ai8, hs1, hs1, dinvb, w2b, b1_2)

    out = pl.pallas_call(
        _gcn_out_body,
        grid=(r,),
        in_specs=[
            pl.BlockSpec((_BM, n), lambda i: (i, 0)),
            pl.BlockSpec((n, dout), lambda i: (0, 0)),
            pl.BlockSpec((_BM, dout), lambda i: (i, 0)),
            pl.BlockSpec((_BM, 8), lambda i: (i, 0)),
            pl.BlockSpec((1, dout), lambda i: (0, 0)),
            pl.BlockSpec((1, _BM), lambda i: (0, i)),
            pl.BlockSpec((dout, dh), lambda i: (0, 0)),
            pl.BlockSpec((1, dh), lambda i: (0, 0)),
            pl.BlockSpec((dh, ncls), lambda i: (0, 0)),
            pl.BlockSpec((1, ncls), lambda i: (0, 0)),
        ],
        out_specs=pl.BlockSpec((g, ncls), lambda i: (0, 0)),
        out_shape=jax.ShapeDtypeStruct((g, ncls), jnp.float32),
        scratch_shapes=[
            pltpu.VMEM((g, dout), jnp.float32),
            pltpu.VMEM((g, 128), jnp.float32),
        ],
        compiler_params=pltpu.CompilerParams(
            dimension_semantics=("arbitrary",),
            vmem_limit_bytes=56 * 1024 * 1024),
        name="gcn_layer2_pool",
    )(ai8, hs2, hs2, dinvb, b2_2, batch2, Wc1, bc1_2, Wc2, bc2_2)
    return out
